# Initial kernel scaffold; baseline (speedup 1.0000x reference)
import jax
import jax.numpy as jnp
from jax.experimental import pallas as pl


def kernel(X, wfa, b2m, w3, b3, w4, b4, w5, b5, se16e, se16o, se8e, se8o, w1t, b1f, w2p, b2f):
    raise NotImplementedError("write your pallas kernel here")


# speedup vs baseline: 2.0615x; 2.0615x over previous
import functools

import jax
import jax.numpy as jnp
from jax.experimental import pallas as pl
from jax.experimental.pallas import tpu as pltpu

_NUM_JOINTS = 25
_NUM_ACTORS = 8
_NUM_CLASSES = 6
_FEAT = 2048

_BB = 8
_BP = 48
_OFF = 8
_SS3 = 496
_SS5 = 288
_M3 = 384
_M5 = 168


def _conv_kernel(xa_ref, wfa_ref, b2m_ref, w3_ref, b3_ref, w4_ref, b4_ref,
                 w5_ref, b5_ref, se16e_ref, se16o_ref, se8e_ref, se8o_ref,
                 out_ref, buf3, buf4, buf5):
    f32 = jnp.float32
    buf3[...] = jnp.zeros_like(buf3)
    buf4[...] = jnp.zeros_like(buf4)
    buf5[...] = jnp.zeros_like(buf5)

    for s in range(_BB):
        acc_a = jnp.zeros((_NUM_ACTORS, 32, 32), f32) + b2m_ref[...][None]
        for kh in range(3):
            for kind in range(2):
                wv = wfa_ref[kh, kind]
                xrow = xa_ref[s, :, kind * 34 + kh: kind * 34 + kh + 32]
                acc_a = acc_a + wv[None] * xrow[:, None, :]
        base = s * _SS3
        for w in range(_NUM_ACTORS):
            a = acc_a[w]
            r0 = base + (w + 1) * _BP + _OFF
            buf3[r0 + 1: r0 + 33, 0:32] = a
            buf3[r0: r0 + 32, 32:64] = a
            buf3[r0 - 1: r0 + 31, 64:96] = a

    for s in range(_BB):
        base = s * _SS3
        acc3 = jnp.dot(buf3[base + _OFF: base + _OFF + _M3, :], w3_ref[0],
                       preferred_element_type=f32)
        acc3 = acc3 + jnp.dot(
            buf3[base + _OFF + _BP: base + _OFF + _BP + _M3, :], w3_ref[1],
            preferred_element_type=f32)
        acc3 = acc3 + jnp.dot(
            buf3[base + _OFF + 2 * _BP: base + _OFF + 2 * _BP + _M3, :],
            w3_ref[2], preferred_element_type=f32)
        acc3 = acc3 + b3_ref[...]
        for w in range(_NUM_ACTORS):
            a3 = acc3[48 * w: 48 * w + 32, :]
            r0 = base + (w + 1) * _BP + _OFF
            buf4[r0 + 1: r0 + 33, 0:64] = a3
            buf4[r0: r0 + 32, 64:128] = a3
            buf4[r0 - 1: r0 + 31, 128:192] = a3

    mcat = []
    for s in range(_BB):
        base = s * _SS3
        acc4 = jnp.dot(buf4[base + _OFF: base + _OFF + _M3, :], w4_ref[0],
                       preferred_element_type=f32)
        acc4 = acc4 + jnp.dot(
            buf4[base + _OFF + _BP: base + _OFF + _BP + _M3, :], w4_ref[1],
            preferred_element_type=f32)
        acc4 = acc4 + jnp.dot(
            buf4[base + _OFF + 2 * _BP: base + _OFF + 2 * _BP + _M3, :],
            w4_ref[2], preferred_element_type=f32)
        acc4 = acc4 + b4_ref[...]
        for a2 in range(4):
            mcat.append(jnp.maximum(acc4[96 * a2: 96 * a2 + 32, :],
                                    acc4[96 * a2 + 48: 96 * a2 + 80, :]))
    mcat = jnp.concatenate(mcat, axis=1)
    p4a = jnp.maximum(
        jnp.dot(se16e_ref[...], mcat, preferred_element_type=f32),
        jnp.dot(se16o_ref[...], mcat, preferred_element_type=f32))
    for s in range(_BB):
        base5 = s * _SS5
        for a2 in range(4):
            c0 = (4 * s + a2) * 32
            p4 = p4a[:, c0: c0 + 32]
            r0 = base5 + (a2 + 1) * _BP + _OFF
            buf5[r0 + 1: r0 + 17, 0:32] = p4
            buf5[r0: r0 + 16, 32:64] = p4
            buf5[r0 - 1: r0 + 15, 64:96] = p4

    m5cat = []
    for s in range(_BB):
        base5 = s * _SS5
        acc5 = jnp.dot(buf5[base5 + _OFF: base5 + _OFF + _M5, :], w5_ref[0],
                       preferred_element_type=f32)
        acc5 = acc5 + jnp.dot(
            buf5[base5 + _OFF + _BP: base5 + _OFF + _BP + _M5, :], w5_ref[1],
            preferred_element_type=f32)
        acc5 = acc5 + jnp.dot(
            buf5[base5 + _OFF + 2 * _BP: base5 + _OFF + 2 * _BP + _M5, :],
            w5_ref[2], preferred_element_type=f32)
        acc5 = acc5 + b5_ref[...]
        for w2 in range(2):
            m5cat.append(jnp.maximum(acc5[96 * w2: 96 * w2 + 16, :],
                                     acc5[96 * w2 + 48: 96 * w2 + 64, :]))
    m5cat = jnp.concatenate(m5cat, axis=1)
    p5a = jnp.maximum(
        jnp.dot(se8e_ref[...], m5cat, preferred_element_type=f32),
        jnp.dot(se8o_ref[...], m5cat, preferred_element_type=f32))
    p5a = jnp.maximum(p5a, 0.0)
    for s in range(_BB):
        for w2 in range(2):
            c0 = (2 * s + w2) * 128
            out_ref[s, w2 * 8: w2 * 8 + 8, :] = p5a[:, c0: c0 + 128]


def _fc_head_kernel(x_ref, w1_ref, b1_ref, w2_ref, b2_ref, o_ref):
    h = jnp.dot(x_ref[...], w1_ref[...], preferred_element_type=jnp.float32)
    h = h + b1_ref[...]
    y = jnp.dot(h, w2_ref[...], preferred_element_type=jnp.float32)
    o_ref[...] = y + b2_ref[...]


def _conv_features(xa, wfa, b2m, w3, b3, w4, b4, w5, b5,
                   se16e, se16o, se8e, se8o):
    Bp = xa.shape[0]
    return pl.pallas_call(
        _conv_kernel,
        out_shape=jax.ShapeDtypeStruct((Bp, 16, 128), jnp.float32),
        grid=(Bp // _BB,),
        in_specs=[
            pl.BlockSpec((_BB, _NUM_ACTORS, 68), lambda i: (i, 0, 0)),
            pl.BlockSpec((3, 2, 32, 1), lambda i: (0, 0, 0, 0)),
            pl.BlockSpec((32, 32), lambda i: (0, 0)),
            pl.BlockSpec((3, 96, 64), lambda i: (0, 0, 0)),
            pl.BlockSpec((1, 64), lambda i: (0, 0)),
            pl.BlockSpec((3, 192, 32), lambda i: (0, 0, 0)),
            pl.BlockSpec((1, 32), lambda i: (0, 0)),
            pl.BlockSpec((3, 96, 128), lambda i: (0, 0, 0)),
            pl.BlockSpec((1, 128), lambda i: (0, 0)),
            pl.BlockSpec((16, 32), lambda i: (0, 0)),
            pl.BlockSpec((16, 32), lambda i: (0, 0)),
            pl.BlockSpec((8, 16), lambda i: (0, 0)),
            pl.BlockSpec((8, 16), lambda i: (0, 0)),
        ],
        out_specs=pl.BlockSpec((_BB, 16, 128), lambda i: (i, 0, 0)),
        scratch_shapes=[
            pltpu.VMEM((_BB * _SS3, 96), jnp.float32),
            pltpu.VMEM((_BB * _SS3, 192), jnp.float32),
            pltpu.VMEM((_BB * _SS5, 96), jnp.float32),
        ],
        compiler_params=pltpu.CompilerParams(dimension_semantics=("parallel",)),
    )(xa, wfa, b2m, w3, b3, w4, b4, w5, b5, se16e, se16o, se8e, se8o)


def _fc_head(person, w1t, b1f, w2p, b2f):
    Bp = person.shape[0]
    bm = next(d for d in (256, 128, 64, 32, 16, 8) if Bp % d == 0)
    return pl.pallas_call(
        _fc_head_kernel,
        out_shape=jax.ShapeDtypeStruct((Bp, 128), jnp.float32),
        grid=(Bp // bm,),
        in_specs=[
            pl.BlockSpec((bm, _FEAT), lambda i: (i, 0)),
            pl.BlockSpec((_FEAT, 256), lambda i: (0, 0)),
            pl.BlockSpec((1, 256), lambda i: (0, 0)),
            pl.BlockSpec((256, 128), lambda i: (0, 0)),
            pl.BlockSpec((1, 128), lambda i: (0, 0)),
        ],
        out_specs=pl.BlockSpec((bm, 128), lambda i: (i, 0)),
        compiler_params=pltpu.CompilerParams(dimension_semantics=("parallel",)),
    )(person, w1t, b1f, w2p, b2f)


@jax.jit
def _forward(X, wfa, b2m, w3, b3, w4, b4, w5, b5,
             se16e, se16o, se8e, se8o, w1t, b1f, w2p, b2f):
    x = X.reshape(-1, 2, _NUM_JOINTS, _NUM_ACTORS).astype(jnp.float32)
    B = x.shape[0]
    Bp = ((B + _BB - 1) // _BB) * _BB
    xt = jnp.transpose(x, (0, 3, 1, 2))
    xt = jnp.pad(xt, ((0, Bp - B), (0, 0), (0, 0), (1, 8)))
    xa = xt.reshape(Bp, _NUM_ACTORS, 68)

    feats = _conv_features(xa, wfa, b2m, w3, b3, w4, b4, w5, b5,
                           se16e, se16o, se8e, se8o)
    person = feats.reshape(Bp, _FEAT)
    out = _fc_head(person, w1t, b1f, w2p, b2f)
    return out[:B, :_NUM_CLASSES]


def kernel(X, wfa, b2m, w3, b3, w4, b4, w5, b5,
           se16e, se16o, se8e, se8o, w1t, b1f, w2p, b2f):
    return _forward(X, wfa, b2m, w3, b3, w4, b4, w5, b5,
                    se16e, se16o, se8e, se8o, w1t, b1f, w2p, b2f)
